# baseline (speedup 1.0000x reference)
import dataclasses
import functools

import jax
import jax.numpy as jnp
from jax import lax
from jax.experimental import pallas as pl
from jax.experimental.pallas import tpu as pltpu
from jax.experimental.pallas import tpu_sc as plsc

NC = 2
NS = 16
NW = NC * NS
LANES = 16
PACK = 4
IDX_CHUNK = 128


def _sc_compiler_params():
    cp = pltpu.CompilerParams(use_tc_tiling_on_sc=True)
    if "needs_layout_passes" in pltpu.CompilerParams.__dataclass_fields__:
        cp = dataclasses.replace(cp, needs_layout_passes=False)
    return cp


def _sc_partials(label, featT, table, b, d):
    b_per_w = b // NW
    n_chunks = b_per_w // IDX_CHUNK
    wide = PACK * d
    mesh = plsc.VectorSubcoreMesh(core_axis_name="c", subcore_axis_name="s")

    @functools.partial(
        pl.kernel,
        mesh=mesh,
        compiler_params=_sc_compiler_params(),
        out_type=jax.ShapeDtypeStruct((NW, LANES), jnp.float32),
        scratch_types=[
            pltpu.VMEM((b_per_w,), jnp.int32),
            pltpu.VMEM((n_chunks, IDX_CHUNK), jnp.int32),
            pltpu.VMEM((b_per_w,), jnp.int32),
            pltpu.VMEM((b_per_w, wide), jnp.float32),
            pltpu.VMEM((d, b_per_w), jnp.float32),
            pltpu.VMEM((LANES,), jnp.float32),
            pltpu.SemaphoreType.DMA,
            pltpu.SemaphoreType.DMA,
        ],
    )
    def k(label_hbm, featT_hbm, table_hbm, out_hbm,
          lab_v, idx_v, sel_v, rows_v, featT_v, acc_v, gsem, fsem):
        wid = lax.axis_index("s") * NC + lax.axis_index("c")
        base = wid * b_per_w

        pltpu.sync_copy(label_hbm.at[pl.ds(base, b_per_w)], lab_v)
        fcp = pltpu.async_copy(
            featT_hbm.at[:, pl.ds(base, b_per_w)], featT_v, fsem)

        for kk in range(b_per_w // LANES):
            lv = lab_v[pl.ds(kk * LANES, LANES)]
            row = lax.shift_right_logical(lv, 2)
            sel = lax.shift_left(jnp.bitwise_and(lv, 3), 5)
            idx_v[kk // (IDX_CHUNK // LANES),
                  pl.ds((kk % (IDX_CHUNK // LANES)) * LANES, LANES)] = row
            sel_v[pl.ds(kk * LANES, LANES)] = sel

        copies = []
        for j in range(n_chunks):
            copies.append(pltpu.async_copy(
                table_hbm.at[idx_v.at[j]],
                rows_v.at[pl.ds(j * IDX_CHUNK, IDX_CHUNK)],
                gsem))
        fcp.wait()
        for c in copies:
            c.wait()

        acc_v[...] = jnp.zeros((LANES,), jnp.float32)
        lane_iota = lax.iota(jnp.int32, LANES)

        @pl.loop(0, b_per_w // LANES)
        def _(c):
            cbase = c * LANES
            row_idx = lane_iota + cbase
            col0 = sel_v[pl.ds(cbase, LANES)]
            acc = acc_v[...]
            for f in range(d):
                g = plsc.load_gather(rows_v, [row_idx, col0 + f])
                dv = featT_v[f, pl.ds(cbase, LANES)] - g
                acc = acc + dv * dv
            acc_v[...] = acc

        pltpu.sync_copy(acc_v, out_hbm.at[wid])

    return k(label, featT, table)


def _tc_reduce(partials, scale):
    def body(x_ref, o_ref):
        o_ref[0, 0] = jnp.sum(x_ref[...]) * scale

    return pl.pallas_call(
        body,
        out_shape=jax.ShapeDtypeStruct((1, 1), jnp.float32),
        out_specs=pl.BlockSpec(memory_space=pltpu.SMEM),
    )(partials)


def kernel(label, feat, centers):
    b, d = feat.shape
    n, _ = centers.shape
    label = label.astype(jnp.int32)
    featT = feat.T
    table = centers.reshape(n // PACK, PACK * d)
    partials = _sc_partials(label, featT, table, b, d)
    out = _tc_reduce(partials, 0.5 / b)
    return out.reshape(())
